# baseline (speedup 1.0000x reference)
import jax
import jax.numpy as jnp
from jax import lax
from jax.experimental import pallas as pl
from jax.experimental.pallas import tpu as pltpu
from jax.experimental.pallas import tpu_sc as plsc

N = 100000
E = 1600000
DF, DH, DO = 128, 16, 2
NC, NS = 2, 16
NW = NC * NS
BATCH = 125
K = 5
ITERS = 80
ROWS_W = K * ITERS
EROWS = NW * ROWS_W
N_ACC = 100864
STRIPE = N_ACC // NS
WR = N_ACC // NW
CHUNKS = (1024, 1024, 1024, 80)
BN = 2000
NB = N // BN

_MESH = plsc.VectorSubcoreMesh(core_axis_name="c", subcore_axis_name="s")
_SC_PARAMS = pltpu.CompilerParams(use_tc_tiling_on_sc=False)


def _wid():
    return lax.axis_index("c") * NS + lax.axis_index("s")


def _newton_rsqrt(x):
    i = lax.bitcast_convert_type(x, jnp.int32)
    i = 0x5F3759DF - lax.shift_right_arithmetic(i, 1)
    y = lax.bitcast_convert_type(i, jnp.float32)
    for _ in range(3):
        y = y * (1.5 - 0.5 * x * y * y)
    return y


def _deg_body(ei_hbm, ones_hbm, zeros_hbm, out_hbm, acc, idx, onesv, isem):
    c = lax.axis_index("c")
    s = lax.axis_index("s")
    wid = c * NS + s
    pltpu.sync_copy(ones_hbm, onesv)
    pltpu.sync_copy(zeros_hbm, acc.at[pl.ds(s * STRIPE, STRIPE)])
    plsc.subcore_barrier()
    row0 = wid * ROWS_W
    pltpu.async_copy(ei_hbm.at[1, pl.ds(row0, K)], idx.at[pl.ds(0, K)], isem)

    def body(j, carry):
        base = lax.rem(j, 2) * K
        pltpu.make_async_copy(ei_hbm.at[1, pl.ds(row0, K)],
                              idx.at[pl.ds(base, K)], isem).wait()

        @pl.when(j + 1 < ITERS)
        def _prefetch():
            pltpu.async_copy(ei_hbm.at[1, pl.ds(row0 + (j + 1) * K, K)],
                             idx.at[pl.ds(K - base, K)], isem)

        for t in range(K):
            pltpu.sync_copy(onesv, acc.at[idx.at[base + t]], add=True)
        return carry

    lax.fori_loop(0, ITERS, body, 0)
    plsc.subcore_barrier()
    pltpu.sync_copy(acc.at[pl.ds(s * STRIPE, STRIPE)],
                    out_hbm.at[pl.ds(c * N_ACC + s * STRIPE, STRIPE)])


_deg_call = pl.kernel(
    _deg_body,
    out_type=jax.ShapeDtypeStruct((NC * N_ACC,), jnp.float32),
    compiler_params=_SC_PARAMS,
    mesh=_MESH,
    scratch_types=[
        pltpu.VMEM_SHARED((N_ACC,), jnp.float32),
        pltpu.VMEM((2 * K, BATCH), jnp.int32),
        pltpu.VMEM((BATCH,), jnp.float32),
        pltpu.SemaphoreType.DMA,
    ],
)


def _agg_body(ei_hbm, h_hbm, zeros_hbm, out_hbm,
              acc, sidx, didx, rows, gsem, ssem, isem):
    c = lax.axis_index("c")
    s = lax.axis_index("s")
    wid = c * NS + s
    pltpu.sync_copy(zeros_hbm, acc.at[pl.ds(s * STRIPE, STRIPE)])
    plsc.subcore_barrier()
    row0 = wid * ROWS_W
    pltpu.async_copy(ei_hbm.at[0, pl.ds(row0, K)], sidx.at[pl.ds(0, K)], isem)
    pltpu.async_copy(ei_hbm.at[1, pl.ds(row0, K)], didx.at[pl.ds(0, K)], isem)

    def body(j, carry):
        base = lax.rem(j, 2) * K
        nbase = K - base
        pltpu.make_async_copy(ei_hbm.at[0, pl.ds(row0, K)],
                              sidx.at[pl.ds(base, K)], isem).wait()
        pltpu.make_async_copy(ei_hbm.at[1, pl.ds(row0, K)],
                              didx.at[pl.ds(base, K)], isem).wait()

        @pl.when(j + 1 < ITERS)
        def _prefetch():
            r = row0 + (j + 1) * K
            pltpu.async_copy(ei_hbm.at[0, pl.ds(r, K)],
                             sidx.at[pl.ds(nbase, K)], isem)
            pltpu.async_copy(ei_hbm.at[1, pl.ds(r, K)],
                             didx.at[pl.ds(nbase, K)], isem)

        @pl.when(j >= 2)
        def _drain():
            for t in range(K):
                pltpu.make_async_copy(h_hbm.at[pl.ds(0, BATCH)],
                                      rows.at[0], ssem).wait()

        gh = [pltpu.async_copy(h_hbm.at[sidx.at[base + t]],
                               rows.at[base + t], gsem)
              for t in range(K)]
        for t in range(K):
            gh[t].wait()
            pltpu.async_copy(rows.at[base + t], acc.at[didx.at[base + t]],
                             ssem, add=True)
        return carry

    lax.fori_loop(0, ITERS, body, 0)
    for t in range(2 * K):
        pltpu.make_async_copy(h_hbm.at[pl.ds(0, BATCH)],
                              rows.at[0], ssem).wait()
    plsc.subcore_barrier()
    pltpu.sync_copy(acc.at[pl.ds(s * STRIPE, STRIPE)],
                    out_hbm.at[pl.ds(c * N_ACC + s * STRIPE, STRIPE)])


_agg16 = pl.kernel(
    _agg_body,
    out_type=jax.ShapeDtypeStruct((NC * N_ACC, DH), jnp.float32),
    compiler_params=_SC_PARAMS,
    mesh=_MESH,
    scratch_types=[
        pltpu.VMEM_SHARED((N_ACC, DH), jnp.float32),
        pltpu.VMEM((2 * K, BATCH), jnp.int32),
        pltpu.VMEM((2 * K, BATCH), jnp.int32),
        pltpu.VMEM((2 * K, BATCH, DH), jnp.float32),
        pltpu.SemaphoreType.DMA,
        pltpu.SemaphoreType.DMA,
        pltpu.SemaphoreType.DMA,
    ],
)


def _dinv_chunk(dp_hbm, d0b, d1b, r, ch):
    pltpu.sync_copy(dp_hbm.at[pl.ds(r, ch)], d0b.at[pl.ds(0, ch)])
    pltpu.sync_copy(dp_hbm.at[pl.ds(N_ACC + r, ch)], d1b.at[pl.ds(0, ch)])


def _scale_body(h_hbm, dp_hbm, out_hbm, hb, ob, d0b, d1b):
    r0 = _wid() * WR
    off = 0
    for ch in CHUNKS:
        r = r0 + off
        pltpu.sync_copy(h_hbm.at[pl.ds(r, ch)], hb.at[pl.ds(0, ch)])
        _dinv_chunk(dp_hbm, d0b, d1b, r, ch)

        def grp(g, carry):
            degv = d0b[pl.ds(g * 16, 16)] + d1b[pl.ds(g * 16, 16)] + 1.0
            dv = _newton_rsqrt(degv)
            for j in range(16):
                ob[g * 16 + j, :] = hb[g * 16 + j, :] * dv[j]
            return carry

        lax.fori_loop(0, ch // 16, grp, 0)
        pltpu.sync_copy(ob.at[pl.ds(0, ch)], out_hbm.at[pl.ds(r, ch)])
        off += ch


_scale_call = pl.kernel(
    _scale_body,
    out_type=jax.ShapeDtypeStruct((N_ACC, DH), jnp.float32),
    compiler_params=_SC_PARAMS,
    mesh=_MESH,
    scratch_types=[
        pltpu.VMEM((1024, DH), jnp.float32),
        pltpu.VMEM((1024, DH), jnp.float32),
        pltpu.VMEM((1024,), jnp.float32),
        pltpu.VMEM((1024,), jnp.float32),
    ],
)


def _mid_body(z_hbm, hp_hbm, dp_hbm, b1_hbm, out_hbm,
              z0b, z1b, hb, ob, d0b, d1b, b1v):
    r0 = _wid() * WR
    pltpu.sync_copy(b1_hbm, b1v)
    off = 0
    for ch in CHUNKS:
        r = r0 + off
        pltpu.sync_copy(z_hbm.at[pl.ds(r, ch)], z0b.at[pl.ds(0, ch)])
        pltpu.sync_copy(z_hbm.at[pl.ds(N_ACC + r, ch)], z1b.at[pl.ds(0, ch)])
        pltpu.sync_copy(hp_hbm.at[pl.ds(r, ch)], hb.at[pl.ds(0, ch)])
        _dinv_chunk(dp_hbm, d0b, d1b, r, ch)
        bv = b1v[...]

        def grp(g, carry):
            degv = d0b[pl.ds(g * 16, 16)] + d1b[pl.ds(g * 16, 16)] + 1.0
            dv = _newton_rsqrt(degv)
            for j in range(16):
                i = g * 16 + j
                z = z0b[i, :] + z1b[i, :] + hb[i, :]
                o1 = jnp.maximum(z * dv[j] + bv, 0.0)
                ob[i, :] = o1 * dv[j]
            return carry

        lax.fori_loop(0, ch // 16, grp, 0)
        pltpu.sync_copy(ob.at[pl.ds(0, ch)], out_hbm.at[pl.ds(r, ch)])
        off += ch


_mid_call = pl.kernel(
    _mid_body,
    out_type=jax.ShapeDtypeStruct((N_ACC, DH), jnp.float32),
    compiler_params=_SC_PARAMS,
    mesh=_MESH,
    scratch_types=[
        pltpu.VMEM((1024, DH), jnp.float32),
        pltpu.VMEM((1024, DH), jnp.float32),
        pltpu.VMEM((1024, DH), jnp.float32),
        pltpu.VMEM((1024, DH), jnp.float32),
        pltpu.VMEM((1024,), jnp.float32),
        pltpu.VMEM((1024,), jnp.float32),
        pltpu.VMEM((DH,), jnp.float32),
    ],
)


def _zz_body(z_hbm, op_hbm, dp_hbm, out_hbm, z0b, z1b, hb, ob, d0b, d1b):
    r0 = _wid() * WR
    off = 0
    for ch in CHUNKS:
        r = r0 + off
        pltpu.sync_copy(z_hbm.at[pl.ds(r, ch)], z0b.at[pl.ds(0, ch)])
        pltpu.sync_copy(z_hbm.at[pl.ds(N_ACC + r, ch)], z1b.at[pl.ds(0, ch)])
        pltpu.sync_copy(op_hbm.at[pl.ds(r, ch)], hb.at[pl.ds(0, ch)])
        _dinv_chunk(dp_hbm, d0b, d1b, r, ch)

        def grp(g, carry):
            degv = d0b[pl.ds(g * 16, 16)] + d1b[pl.ds(g * 16, 16)] + 1.0
            dv = _newton_rsqrt(degv)
            for j in range(16):
                i = g * 16 + j
                zz = z0b[i, :] + z1b[i, :] + hb[i, :]
                ob[i, :] = zz * dv[j]
            return carry

        lax.fori_loop(0, ch // 16, grp, 0)
        pltpu.sync_copy(ob.at[pl.ds(0, ch)], out_hbm.at[pl.ds(r, ch)])
        off += ch


_zz_call = pl.kernel(
    _zz_body,
    out_type=jax.ShapeDtypeStruct((N_ACC, DH), jnp.float32),
    compiler_params=_SC_PARAMS,
    mesh=_MESH,
    scratch_types=[
        pltpu.VMEM((1024, DH), jnp.float32),
        pltpu.VMEM((1024, DH), jnp.float32),
        pltpu.VMEM((1024, DH), jnp.float32),
        pltpu.VMEM((1024, DH), jnp.float32),
        pltpu.VMEM((1024,), jnp.float32),
        pltpu.VMEM((1024,), jnp.float32),
    ],
)


def _mm1_body(x_ref, w_ref, o_ref):
    o_ref[...] = jnp.dot(x_ref[...], w_ref[...],
                         preferred_element_type=jnp.float32)


_mm1 = pl.pallas_call(
    _mm1_body,
    grid=(NB,),
    in_specs=[pl.BlockSpec((BN, DF), lambda i: (i, 0)),
              pl.BlockSpec((DF, DH), lambda i: (0, 0))],
    out_specs=pl.BlockSpec((BN, DH), lambda i: (i, 0)),
    out_shape=jax.ShapeDtypeStruct((N_ACC, DH), jnp.float32),
)


PKROWS = N_ACC * DH // 128
PKB = PKROWS // 8


def _fin_body(zz_ref, b2_ref, w2_ref, p_ref, o_ref):
    t = (jnp.dot(zz_ref[...], w2_ref[...], preferred_element_type=jnp.float32)
         + b2_ref[...])
    t = t - jnp.max(t, axis=1, keepdims=True)
    s = jnp.dot(jnp.exp(t), p_ref[...], preferred_element_type=jnp.float32)
    o_ref[...] = t - jnp.log(s)


_fin = pl.pallas_call(
    _fin_body,
    grid=(8,),
    in_specs=[pl.BlockSpec((PKB, 128), lambda i: (i, 0)),
              pl.BlockSpec((1, DH), lambda i: (0, 0)),
              pl.BlockSpec((128, DH), lambda i: (0, 0)),
              pl.BlockSpec((DH, DH), lambda i: (0, 0))],
    out_specs=pl.BlockSpec((PKB, DH), lambda i: (i, 0)),
    out_shape=jax.ShapeDtypeStruct((PKROWS, DH), jnp.float32),
)


def kernel(x, edge_index, W1, b1, W2, b2):
    ei3 = edge_index.reshape(2, EROWS, BATCH)
    ones125 = jnp.ones((BATCH,), jnp.float32)
    zer1 = jnp.zeros((STRIPE,), jnp.float32)
    zer16 = jnp.zeros((STRIPE, DH), jnp.float32)

    dpart = _deg_call(ei3, ones125, zer1)
    h1 = _mm1(x, W1)
    h1p = _scale_call(h1, dpart)
    z1 = _agg16(ei3, h1p, zer16)
    o1p = _mid_call(z1, h1p, dpart, b1)
    z2 = _agg16(ei3, o1p, zer16)
    zz = _zz_call(z2, o1p, dpart)

    w2big = jnp.kron(jnp.eye(8, dtype=jnp.float32), W2)
    b2big = jnp.tile(b2, 8)[None, :]
    pair = jnp.kron(jnp.eye(8, dtype=jnp.float32),
                    jnp.ones((DO, DO), jnp.float32))
    opk = _fin(zz.reshape(PKROWS, 128), b2big, w2big, pair)
    return opk.reshape(N_ACC, DO)[:N]


# speedup vs baseline: 1.0705x; 1.0705x over previous
import jax
import jax.numpy as jnp
from jax import lax
from jax.experimental import pallas as pl
from jax.experimental.pallas import tpu as pltpu
from jax.experimental.pallas import tpu_sc as plsc

N = 100000
E = 1600000
DF, DH, DO = 128, 16, 2
NC, NS = 2, 16
NW = NC * NS
BATCH = 125
K = 10
ITERS = 40
ROWS_W = K * ITERS
EROWS = NW * ROWS_W
N_ACC = 100864
STRIPE = N_ACC // NS
WR = N_ACC // NW
CHUNKS = (1024, 1024, 1024, 80)
BN = 2000
NB = N // BN

_MESH = plsc.VectorSubcoreMesh(core_axis_name="c", subcore_axis_name="s")
_SC_PARAMS = pltpu.CompilerParams(use_tc_tiling_on_sc=False)


def _wid():
    return lax.axis_index("c") * NS + lax.axis_index("s")


def _newton_rsqrt(x):
    i = lax.bitcast_convert_type(x, jnp.int32)
    i = 0x5F3759DF - lax.shift_right_arithmetic(i, 1)
    y = lax.bitcast_convert_type(i, jnp.float32)
    for _ in range(3):
        y = y * (1.5 - 0.5 * x * y * y)
    return y


def _deg_body(ei_hbm, ones_hbm, zeros_hbm, out_hbm, acc, idx, onesv, isem):
    c = lax.axis_index("c")
    s = lax.axis_index("s")
    wid = c * NS + s
    pltpu.sync_copy(ones_hbm, onesv)
    pltpu.sync_copy(zeros_hbm, acc.at[pl.ds(s * STRIPE, STRIPE)])
    plsc.subcore_barrier()
    row0 = wid * ROWS_W
    pltpu.async_copy(ei_hbm.at[1, pl.ds(row0, K)], idx.at[pl.ds(0, K)], isem)

    def body(j, carry):
        base = lax.rem(j, 2) * K
        pltpu.make_async_copy(ei_hbm.at[1, pl.ds(row0, K)],
                              idx.at[pl.ds(base, K)], isem).wait()

        @pl.when(j + 1 < ITERS)
        def _prefetch():
            pltpu.async_copy(ei_hbm.at[1, pl.ds(row0 + (j + 1) * K, K)],
                             idx.at[pl.ds(K - base, K)], isem)

        for t in range(K):
            pltpu.sync_copy(onesv, acc.at[idx.at[base + t]], add=True)
        return carry

    lax.fori_loop(0, ITERS, body, 0)
    plsc.subcore_barrier()
    pltpu.sync_copy(acc.at[pl.ds(s * STRIPE, STRIPE)],
                    out_hbm.at[pl.ds(c * N_ACC + s * STRIPE, STRIPE)])


_deg_call = pl.kernel(
    _deg_body,
    out_type=jax.ShapeDtypeStruct((NC * N_ACC,), jnp.float32),
    compiler_params=_SC_PARAMS,
    mesh=_MESH,
    scratch_types=[
        pltpu.VMEM_SHARED((N_ACC,), jnp.float32),
        pltpu.VMEM((2 * K, BATCH), jnp.int32),
        pltpu.VMEM((BATCH,), jnp.float32),
        pltpu.SemaphoreType.DMA,
    ],
)


def _agg_body(ei_hbm, h_hbm, zeros_hbm, out_hbm,
              acc, sidx, didx, rows, gsem, ssem, isem):
    c = lax.axis_index("c")
    s = lax.axis_index("s")
    wid = c * NS + s
    pltpu.sync_copy(zeros_hbm, acc.at[pl.ds(s * STRIPE, STRIPE)])
    plsc.subcore_barrier()
    row0 = wid * ROWS_W
    pltpu.async_copy(ei_hbm.at[0, pl.ds(row0, K)], sidx.at[pl.ds(0, K)], isem)
    pltpu.async_copy(ei_hbm.at[1, pl.ds(row0, K)], didx.at[pl.ds(0, K)], isem)

    def body(j, carry):
        base = lax.rem(j, 2) * K
        nbase = K - base
        pltpu.make_async_copy(ei_hbm.at[0, pl.ds(row0, K)],
                              sidx.at[pl.ds(base, K)], isem).wait()
        pltpu.make_async_copy(ei_hbm.at[1, pl.ds(row0, K)],
                              didx.at[pl.ds(base, K)], isem).wait()

        @pl.when(j + 1 < ITERS)
        def _prefetch():
            r = row0 + (j + 1) * K
            pltpu.async_copy(ei_hbm.at[0, pl.ds(r, K)],
                             sidx.at[pl.ds(nbase, K)], isem)
            pltpu.async_copy(ei_hbm.at[1, pl.ds(r, K)],
                             didx.at[pl.ds(nbase, K)], isem)

        @pl.when(j > 0)
        def _drain():
            for t in range(K):
                pltpu.make_async_copy(h_hbm.at[pl.ds(0, BATCH)],
                                      rows.at[t], ssem).wait()

        gh = [pltpu.async_copy(h_hbm.at[sidx.at[base + t]], rows.at[t], gsem)
              for t in range(K)]
        for t in range(K):
            gh[t].wait()
            pltpu.async_copy(rows.at[t], acc.at[didx.at[base + t]], ssem,
                             add=True)
        return carry

    lax.fori_loop(0, ITERS, body, 0)
    for t in range(K):
        pltpu.make_async_copy(h_hbm.at[pl.ds(0, BATCH)],
                              rows.at[t], ssem).wait()
    plsc.subcore_barrier()
    pltpu.sync_copy(acc.at[pl.ds(s * STRIPE, STRIPE)],
                    out_hbm.at[pl.ds(c * N_ACC + s * STRIPE, STRIPE)])


_agg16 = pl.kernel(
    _agg_body,
    out_type=jax.ShapeDtypeStruct((NC * N_ACC, DH), jnp.float32),
    compiler_params=_SC_PARAMS,
    mesh=_MESH,
    scratch_types=[
        pltpu.VMEM_SHARED((N_ACC, DH), jnp.float32),
        pltpu.VMEM((2 * K, BATCH), jnp.int32),
        pltpu.VMEM((2 * K, BATCH), jnp.int32),
        pltpu.VMEM((K, BATCH, DH), jnp.float32),
        pltpu.SemaphoreType.DMA,
        pltpu.SemaphoreType.DMA,
        pltpu.SemaphoreType.DMA,
    ],
)


def _dinv_chunk(dp_hbm, d0b, d1b, r, ch):
    pltpu.sync_copy(dp_hbm.at[pl.ds(r, ch)], d0b.at[pl.ds(0, ch)])
    pltpu.sync_copy(dp_hbm.at[pl.ds(N_ACC + r, ch)], d1b.at[pl.ds(0, ch)])


def _scale_body(h_hbm, dp_hbm, out_hbm, hb, ob, d0b, d1b):
    r0 = _wid() * WR
    off = 0
    for ch in CHUNKS:
        r = r0 + off
        pltpu.sync_copy(h_hbm.at[pl.ds(r, ch)], hb.at[pl.ds(0, ch)])
        _dinv_chunk(dp_hbm, d0b, d1b, r, ch)

        def grp(g, carry):
            degv = d0b[pl.ds(g * 16, 16)] + d1b[pl.ds(g * 16, 16)] + 1.0
            dv = _newton_rsqrt(degv)
            for j in range(16):
                ob[g * 16 + j, :] = hb[g * 16 + j, :] * dv[j]
            return carry

        lax.fori_loop(0, ch // 16, grp, 0)
        pltpu.sync_copy(ob.at[pl.ds(0, ch)], out_hbm.at[pl.ds(r, ch)])
        off += ch


_scale_call = pl.kernel(
    _scale_body,
    out_type=jax.ShapeDtypeStruct((N_ACC, DH), jnp.float32),
    compiler_params=_SC_PARAMS,
    mesh=_MESH,
    scratch_types=[
        pltpu.VMEM((1024, DH), jnp.float32),
        pltpu.VMEM((1024, DH), jnp.float32),
        pltpu.VMEM((1024,), jnp.float32),
        pltpu.VMEM((1024,), jnp.float32),
    ],
)


def _mid_body(z_hbm, hp_hbm, dp_hbm, b1_hbm, out_hbm,
              z0b, z1b, hb, ob, d0b, d1b, b1v):
    r0 = _wid() * WR
    pltpu.sync_copy(b1_hbm, b1v)
    off = 0
    for ch in CHUNKS:
        r = r0 + off
        pltpu.sync_copy(z_hbm.at[pl.ds(r, ch)], z0b.at[pl.ds(0, ch)])
        pltpu.sync_copy(z_hbm.at[pl.ds(N_ACC + r, ch)], z1b.at[pl.ds(0, ch)])
        pltpu.sync_copy(hp_hbm.at[pl.ds(r, ch)], hb.at[pl.ds(0, ch)])
        _dinv_chunk(dp_hbm, d0b, d1b, r, ch)
        bv = b1v[...]

        def grp(g, carry):
            degv = d0b[pl.ds(g * 16, 16)] + d1b[pl.ds(g * 16, 16)] + 1.0
            dv = _newton_rsqrt(degv)
            for j in range(16):
                i = g * 16 + j
                z = z0b[i, :] + z1b[i, :] + hb[i, :]
                o1 = jnp.maximum(z * dv[j] + bv, 0.0)
                ob[i, :] = o1 * dv[j]
            return carry

        lax.fori_loop(0, ch // 16, grp, 0)
        pltpu.sync_copy(ob.at[pl.ds(0, ch)], out_hbm.at[pl.ds(r, ch)])
        off += ch


_mid_call = pl.kernel(
    _mid_body,
    out_type=jax.ShapeDtypeStruct((N_ACC, DH), jnp.float32),
    compiler_params=_SC_PARAMS,
    mesh=_MESH,
    scratch_types=[
        pltpu.VMEM((1024, DH), jnp.float32),
        pltpu.VMEM((1024, DH), jnp.float32),
        pltpu.VMEM((1024, DH), jnp.float32),
        pltpu.VMEM((1024, DH), jnp.float32),
        pltpu.VMEM((1024,), jnp.float32),
        pltpu.VMEM((1024,), jnp.float32),
        pltpu.VMEM((DH,), jnp.float32),
    ],
)


def _zz_body(z_hbm, op_hbm, dp_hbm, out_hbm, z0b, z1b, hb, ob, d0b, d1b):
    r0 = _wid() * WR
    off = 0
    for ch in CHUNKS:
        r = r0 + off
        pltpu.sync_copy(z_hbm.at[pl.ds(r, ch)], z0b.at[pl.ds(0, ch)])
        pltpu.sync_copy(z_hbm.at[pl.ds(N_ACC + r, ch)], z1b.at[pl.ds(0, ch)])
        pltpu.sync_copy(op_hbm.at[pl.ds(r, ch)], hb.at[pl.ds(0, ch)])
        _dinv_chunk(dp_hbm, d0b, d1b, r, ch)

        def grp(g, carry):
            degv = d0b[pl.ds(g * 16, 16)] + d1b[pl.ds(g * 16, 16)] + 1.0
            dv = _newton_rsqrt(degv)
            for j in range(16):
                i = g * 16 + j
                zz = z0b[i, :] + z1b[i, :] + hb[i, :]
                ob[i, :] = zz * dv[j]
            return carry

        lax.fori_loop(0, ch // 16, grp, 0)
        pltpu.sync_copy(ob.at[pl.ds(0, ch)], out_hbm.at[pl.ds(r, ch)])
        off += ch


_zz_call = pl.kernel(
    _zz_body,
    out_type=jax.ShapeDtypeStruct((N_ACC, DH), jnp.float32),
    compiler_params=_SC_PARAMS,
    mesh=_MESH,
    scratch_types=[
        pltpu.VMEM((1024, DH), jnp.float32),
        pltpu.VMEM((1024, DH), jnp.float32),
        pltpu.VMEM((1024, DH), jnp.float32),
        pltpu.VMEM((1024, DH), jnp.float32),
        pltpu.VMEM((1024,), jnp.float32),
        pltpu.VMEM((1024,), jnp.float32),
    ],
)


def _mm1_body(x_ref, w_ref, o_ref):
    o_ref[...] = jnp.dot(x_ref[...], w_ref[...],
                         preferred_element_type=jnp.float32)


_mm1 = pl.pallas_call(
    _mm1_body,
    grid=(NB,),
    in_specs=[pl.BlockSpec((BN, DF), lambda i: (i, 0)),
              pl.BlockSpec((DF, DH), lambda i: (0, 0))],
    out_specs=pl.BlockSpec((BN, DH), lambda i: (i, 0)),
    out_shape=jax.ShapeDtypeStruct((N_ACC, DH), jnp.float32),
)


PKROWS = N_ACC * DH // 128
PKB = PKROWS // 8


def _fin_body(zz_ref, b2_ref, w2_ref, p_ref, o_ref):
    t = (jnp.dot(zz_ref[...], w2_ref[...], preferred_element_type=jnp.float32)
         + b2_ref[...])
    t = t - jnp.max(t, axis=1, keepdims=True)
    s = jnp.dot(jnp.exp(t), p_ref[...], preferred_element_type=jnp.float32)
    o_ref[...] = t - jnp.log(s)


_fin = pl.pallas_call(
    _fin_body,
    grid=(8,),
    in_specs=[pl.BlockSpec((PKB, 128), lambda i: (i, 0)),
              pl.BlockSpec((1, DH), lambda i: (0, 0)),
              pl.BlockSpec((128, DH), lambda i: (0, 0)),
              pl.BlockSpec((DH, DH), lambda i: (0, 0))],
    out_specs=pl.BlockSpec((PKB, DH), lambda i: (i, 0)),
    out_shape=jax.ShapeDtypeStruct((PKROWS, DH), jnp.float32),
)


def kernel(x, edge_index, W1, b1, W2, b2):
    ei3 = edge_index.reshape(2, EROWS, BATCH)
    ones125 = jnp.ones((BATCH,), jnp.float32)
    zer1 = jnp.zeros((STRIPE,), jnp.float32)
    zer16 = jnp.zeros((STRIPE, DH), jnp.float32)

    dpart = _deg_call(ei3, ones125, zer1)
    h1 = _mm1(x, W1)
    h1p = _scale_call(h1, dpart)
    z1 = _agg16(ei3, h1p, zer16)
    o1p = _mid_call(z1, h1p, dpart, b1)
    z2 = _agg16(ei3, o1p, zer16)
    zz = _zz_call(z2, o1p, dpart)

    w2big = jnp.kron(jnp.eye(8, dtype=jnp.float32), W2)
    b2big = jnp.tile(b2, 8)[None, :]
    pair = jnp.kron(jnp.eye(8, dtype=jnp.float32),
                    jnp.ones((DO, DO), jnp.float32))
    opk = _fin(zz.reshape(PKROWS, 128), b2big, w2big, pair)
    return opk.reshape(N_ACC, DO)[:N]
